# baseline (speedup 1.0000x reference)
import functools
import math

import jax
import jax.numpy as jnp
from jax import lax
from jax.experimental import pallas as pl
from jax.experimental.pallas import tpu as pltpu
from jax.experimental.pallas import tpu_sc as plsc

N = 10000
E = 320000
H = 128
R = 64
L = 4
NG = 64
CUT = 6.0
ZMAX = 119

BN = 2000
BE = 512
NW = 32
EPW = E // NW
CH = 40
NCH = EPW // CH
RPS = 624
RPS_LAST = N - 15 * RPS


def _silu(v):
    return v / (1.0 + jnp.exp(-v))


def _embed_body(z_ref, emb_ref, o_ref):
    zcol = z_ref[0]
    onehot = (zcol == lax.broadcasted_iota(jnp.int32, (BN, H), 1))
    o_ref[...] = jnp.dot(onehot.astype(jnp.float32), emb_ref[...],
                         preferred_element_type=jnp.float32,
                         precision=lax.Precision.HIGHEST)


def _embed(z3, emb_pad):
    return pl.pallas_call(
        _embed_body,
        grid=(N // BN,),
        in_specs=[
            pl.BlockSpec((1, BN, 1), lambda i: (i, 0, 0)),
            pl.BlockSpec((H, H), lambda i: (0, 0)),
        ],
        out_specs=pl.BlockSpec((BN, H), lambda i: (i, 0)),
        out_shape=jax.ShapeDtypeStruct((N, H), jnp.float32),
    )(z3, emb_pad)


def _filters_body(d_ref, f1_ref, f1b_ref, f2_ref, f2b_ref, o_ref):
    dist = d_ref[0]
    centers = lax.broadcasted_iota(jnp.int32, (BE, R), 1).astype(jnp.float32) \
        * (CUT / (R - 1))
    gamma = 1.0 / max((CUT / (R - 1)), 1e-6) ** 2
    rbf = jnp.exp(-gamma * (dist - centers) ** 2)
    env = 0.5 * (jnp.clip(jnp.cos(math.pi * dist / CUT), -1.0, 1.0) + 1.0)
    env = env * (dist < CUT).astype(jnp.float32)
    t = jnp.dot(rbf, f1_ref[...], preferred_element_type=jnp.float32)
    t = _silu(t + f1b_ref[...])
    t = jnp.dot(t, f2_ref[...], preferred_element_type=jnp.float32)
    t = t + f2b_ref[...]
    o_ref[...] = t * env


def _filters(d3, f1, f1b, f2, f2b):
    return pl.pallas_call(
        _filters_body,
        grid=(E // BE,),
        in_specs=[
            pl.BlockSpec((1, BE, 1), lambda i: (i, 0, 0)),
            pl.BlockSpec((R, H), lambda i: (0, 0)),
            pl.BlockSpec((1, H), lambda i: (0, 0)),
            pl.BlockSpec((H, H), lambda i: (0, 0)),
            pl.BlockSpec((1, H), lambda i: (0, 0)),
        ],
        out_specs=pl.BlockSpec((BE, H), lambda i: (i, 0)),
        out_shape=jax.ShapeDtypeStruct((E, H), jnp.float32),
    )(d3, f1, f1b, f2, f2b)


def _sc_body(x_hbm, wf_hbm, sd_hbm, zer_hbm, out_hbm,
             idx_v, wf_v, xg_v, agg_sh,
             sem_wf, sem_g, sem_sc):
    cid = lax.axis_index("c")
    sid = lax.axis_index("s")
    wid = cid * 16 + sid
    row0 = sid * RPS

    @pl.when(sid < 15)
    def _z0():
        pltpu.sync_copy(zer_hbm.at[pl.ds(row0, RPS)], agg_sh.at[pl.ds(row0, RPS)])

    @pl.when(sid == 15)
    def _z1():
        pltpu.sync_copy(zer_hbm.at[pl.ds(15 * RPS, RPS_LAST)],
                        agg_sh.at[pl.ds(15 * RPS, RPS_LAST)])

    plsc.subcore_barrier()

    def start_in(c, b):
        base = wid * EPW + c * CH
        pltpu.async_copy(wf_hbm.at[pl.ds(base, CH)], wf_v.at[b], sem_wf)
        pltpu.async_copy(sd_hbm.at[wid, c], idx_v.at[b], sem_wf)

    def wait_in(b):
        pltpu.make_async_copy(wf_hbm.at[pl.ds(0, CH)], wf_v.at[b], sem_wf).wait()
        pltpu.make_async_copy(sd_hbm.at[0, 0], idx_v.at[b], sem_wf).wait()

    def drain_sc(b):
        pltpu.make_async_copy(xg_v.at[b], agg_sh.at[idx_v.at[b, 1]], sem_sc).wait()

    def compute(b):
        def row(e, c2):
            for j in range(H // 16):
                sl = pl.ds(j * 16, 16)
                xg_v[b, e, sl] = xg_v[b, e, sl] * wf_v[b, e, sl]
            return c2

        lax.fori_loop(0, CH, row, 0)

    def chunk_step(c, b, first, last):
        wait_in(b)
        gather = pltpu.async_copy(x_hbm.at[idx_v.at[b, 0]], xg_v.at[b], sem_g)
        if not first:
            drain_sc(1 - b)

        if not last:
            @pl.when(c + 1 < NCH)
            def _pf():
                start_in(c + 1, 1 - b)

        gather.wait()
        compute(b)
        pltpu.async_copy(xg_v.at[b], agg_sh.at[idx_v.at[b, 1]], sem_sc, add=True)

    start_in(0, 0)

    def pair(g, carry):
        c0 = 2 * g

        @pl.when(g == 0)
        def _first():
            chunk_step(0, 0, True, False)

        @pl.when(g > 0)
        def _steady():
            chunk_step(c0, 0, False, False)

        chunk_step(c0 + 1, 1, False, False)
        return carry

    lax.fori_loop(0, NCH // 2, pair, 0)
    drain_sc(1)
    plsc.subcore_barrier()

    @pl.when(sid < 15)
    def _w0():
        pltpu.sync_copy(agg_sh.at[pl.ds(row0, RPS)],
                        out_hbm.at[pl.ds(cid * N + row0, RPS)])

    @pl.when(sid == 15)
    def _w1():
        pltpu.sync_copy(agg_sh.at[pl.ds(15 * RPS, RPS_LAST)],
                        out_hbm.at[pl.ds(cid * N + 15 * RPS, RPS_LAST)])


def _sc_layer(x, wf, sd, zeros_nh):
    mesh = plsc.VectorSubcoreMesh(core_axis_name="c", subcore_axis_name="s",
                                  num_cores=2, num_subcores=16)
    k = pl.kernel(
        _sc_body,
        out_type=jax.ShapeDtypeStruct((2 * N, H), jnp.float32),
        mesh=mesh,
        scratch_types=[
            pltpu.VMEM((2, 2, CH), jnp.int32),
            pltpu.VMEM((2, CH, H), jnp.float32),
            pltpu.VMEM((2, CH, H), jnp.float32),
            pltpu.VMEM_SHARED((N, H), jnp.float32),
            pltpu.SemaphoreType.DMA,
            pltpu.SemaphoreType.DMA,
            pltpu.SemaphoreType.DMA,
        ],
    )
    return k(x, wf, sd, zeros_nh)


def _update_body(x_ref, agg_ref, u1_ref, u1b_ref, u2_ref, u2b_ref, o_ref):
    agg = agg_ref[0] + agg_ref[1]
    t = _silu(jnp.dot(agg, u1_ref[...], preferred_element_type=jnp.float32)
              + u1b_ref[...])
    upd = jnp.dot(t, u2_ref[...], preferred_element_type=jnp.float32) + u2b_ref[...]
    o_ref[...] = x_ref[...] + upd


def _update(x, aggp, u1, u1b, u2, u2b):
    return pl.pallas_call(
        _update_body,
        grid=(N // BN,),
        in_specs=[
            pl.BlockSpec((BN, H), lambda i: (i, 0)),
            pl.BlockSpec((2, BN, H), lambda i: (0, i, 0)),
            pl.BlockSpec((H, H), lambda i: (0, 0)),
            pl.BlockSpec((1, H), lambda i: (0, 0)),
            pl.BlockSpec((H, H), lambda i: (0, 0)),
            pl.BlockSpec((1, H), lambda i: (0, 0)),
        ],
        out_specs=pl.BlockSpec((BN, H), lambda i: (i, 0)),
        out_shape=jax.ShapeDtypeStruct((N, H), jnp.float32),
    )(x, aggp, u1, u1b, u2, u2b)


def _readout_body(b_ref, x_ref, g1_ref, g1b_ref, g2_ref, g2b_ref,
                  h1_ref, h1b_ref, h2_ref, o_ref, ssum_ref, cnt_ref):
    i = pl.program_id(0)
    nblk = pl.num_programs(0)

    @pl.when(i == 0)
    def _init():
        ssum_ref[...] = jnp.zeros_like(ssum_ref)
        cnt_ref[...] = jnp.zeros_like(cnt_ref)

    bcol = b_ref[0]
    onehot = (bcol == lax.broadcasted_iota(jnp.int32, (BN, NG), 1))
    oh = onehot.astype(jnp.float32)
    ssum_ref[...] += lax.dot_general(oh, x_ref[...],
                                     (((0,), (0,)), ((), ())),
                                     preferred_element_type=jnp.float32,
                                     precision=lax.Precision.HIGHEST)
    cnt_ref[...] += lax.dot_general(oh, jnp.ones((BN, H), jnp.float32),
                                    (((0,), (0,)), ((), ())),
                                    preferred_element_type=jnp.float32,
                                    precision=lax.Precision.HIGHEST)

    @pl.when(i == nblk - 1)
    def _fin():
        g = ssum_ref[...] / jnp.maximum(cnt_ref[...], 1.0)
        t = _silu(jnp.dot(g, g1_ref[...], preferred_element_type=jnp.float32)
                  + g1b_ref[...])
        t = jnp.dot(t, g2_ref[...], preferred_element_type=jnp.float32) + g2b_ref[...]
        t = _silu(jnp.dot(t, h1_ref[...], preferred_element_type=jnp.float32)
                  + h1b_ref[...])
        o_ref[...] = jnp.dot(t, h2_ref[...], preferred_element_type=jnp.float32)


def _readout(b3, x, g1, g1b, g2, g2b, h1p, h1bp, h2p):
    return pl.pallas_call(
        _readout_body,
        grid=(N // BN,),
        in_specs=[
            pl.BlockSpec((1, BN, 1), lambda i: (i, 0, 0)),
            pl.BlockSpec((BN, H), lambda i: (i, 0)),
            pl.BlockSpec((H, H), lambda i: (0, 0)),
            pl.BlockSpec((1, H), lambda i: (0, 0)),
            pl.BlockSpec((H, H), lambda i: (0, 0)),
            pl.BlockSpec((1, H), lambda i: (0, 0)),
            pl.BlockSpec((H, H), lambda i: (0, 0)),
            pl.BlockSpec((1, H), lambda i: (0, 0)),
            pl.BlockSpec((H, H), lambda i: (0, 0)),
        ],
        out_specs=pl.BlockSpec((NG, H), lambda i: (0, 0)),
        out_shape=jax.ShapeDtypeStruct((NG, H), jnp.float32),
        scratch_shapes=[
            pltpu.VMEM((NG, H), jnp.float32),
            pltpu.VMEM((NG, H), jnp.float32),
        ],
    )(b3, x, g1, g1b, g2, g2b, h1p, h1bp, h2p)


def kernel(z, edge_index, edge_dist, batch, params):
    p = params
    src = edge_index[0].astype(jnp.int32)
    dst = edge_index[1].astype(jnp.int32)

    emb_pad = jnp.zeros((H, H), jnp.float32).at[:ZMAX].set(p['atom_emb'])
    z3 = z.astype(jnp.int32).reshape(N // BN, BN, 1)
    b3 = batch.astype(jnp.int32).reshape(N // BN, BN, 1)
    d3 = edge_dist.reshape(E // BE, BE, 1)

    sd = jnp.stack([src.reshape(NW, NCH, CH), dst.reshape(NW, NCH, CH)], axis=2)

    x = _embed(z3, emb_pad)
    wfs = [_filters(d3, lp['f1_w'], lp['f1_b'].reshape(1, H),
                    lp['f2_w'], lp['f2_b'].reshape(1, H))
           for lp in p['layers']]
    zeros_nh = jnp.zeros((N, H), jnp.float32)

    for l, lp in enumerate(p['layers']):
        aggp = _sc_layer(x, wfs[l], sd, zeros_nh)
        x = _update(x, aggp.reshape(2, N, H),
                    lp['u1_w'], lp['u1_b'].reshape(1, H),
                    lp['u2_w'], lp['u2_b'].reshape(1, H))

    h1p = jnp.zeros((H, H), jnp.float32).at[:, :H // 2].set(p['h1_w'])
    h1bp = jnp.zeros((1, H), jnp.float32).at[0, :H // 2].set(p['h1_b'])
    h2p = jnp.zeros((H, H), jnp.float32).at[:H // 2, 0].set(p['h2_w'][:, 0])
    out = _readout(b3, x,
                   p['g1_w'], p['g1_b'].reshape(1, H),
                   p['g2_w'], p['g2_b'].reshape(1, H),
                   h1p, h1bp, h2p)
    return out[:, 0] + p['h2_b'][0]


# speedup vs baseline: 1.4545x; 1.4545x over previous
import functools
import math

import jax
import jax.numpy as jnp
from jax import lax
from jax.experimental import pallas as pl
from jax.experimental.pallas import tpu as pltpu
from jax.experimental.pallas import tpu_sc as plsc

N = 10000
E = 320000
H = 128
R = 64
L = 4
NG = 64
CUT = 6.0
ZMAX = 119

BN = 2000
BE = 512
NW = 32
EPW = E // NW
CH = 40
NCH = EPW // CH
RPS = 624
RPS_LAST = N - 15 * RPS


def _silu(v):
    return v / (1.0 + jnp.exp(-v))


def _embed_body(z_ref, emb_ref, o_ref):
    zcol = z_ref[0]
    onehot = (zcol == lax.broadcasted_iota(jnp.int32, (BN, H), 1))
    o_ref[...] = jnp.dot(onehot.astype(jnp.float32), emb_ref[...],
                         preferred_element_type=jnp.float32,
                         precision=lax.Precision.HIGHEST)


def _embed(z3, emb_pad):
    return pl.pallas_call(
        _embed_body,
        grid=(N // BN,),
        in_specs=[
            pl.BlockSpec((1, BN, 1), lambda i: (i, 0, 0)),
            pl.BlockSpec((H, H), lambda i: (0, 0)),
        ],
        out_specs=pl.BlockSpec((BN, H), lambda i: (i, 0)),
        out_shape=jax.ShapeDtypeStruct((N, H), jnp.float32),
    )(z3, emb_pad)


def _filters_body(d_ref, f1_ref, f1b_ref, f2_ref, f2b_ref, *o_refs):
    dist = d_ref[0]
    centers = lax.broadcasted_iota(jnp.int32, (BE, R), 1).astype(jnp.float32) \
        * (CUT / (R - 1))
    gamma = 1.0 / max((CUT / (R - 1)), 1e-6) ** 2
    rbf = jnp.exp(-gamma * (dist - centers) ** 2)
    env = 0.5 * (jnp.clip(jnp.cos(math.pi * dist / CUT), -1.0, 1.0) + 1.0)
    env = env * (dist < CUT).astype(jnp.float32)
    for l in range(L):
        t = jnp.dot(rbf, f1_ref[l], preferred_element_type=jnp.float32)
        t = _silu(t + f1b_ref[l])
        t = jnp.dot(t, f2_ref[l], preferred_element_type=jnp.float32)
        t = t + f2b_ref[l]
        o_refs[l][...] = t * env


def _filters(d3, f1s, f1bs, f2s, f2bs):
    return pl.pallas_call(
        _filters_body,
        grid=(E // BE,),
        in_specs=[
            pl.BlockSpec((1, BE, 1), lambda i: (i, 0, 0)),
            pl.BlockSpec((L, R, H), lambda i: (0, 0, 0)),
            pl.BlockSpec((L, 1, H), lambda i: (0, 0, 0)),
            pl.BlockSpec((L, H, H), lambda i: (0, 0, 0)),
            pl.BlockSpec((L, 1, H), lambda i: (0, 0, 0)),
        ],
        out_specs=[pl.BlockSpec((BE, H), lambda i: (i, 0)) for _ in range(L)],
        out_shape=[jax.ShapeDtypeStruct((E, H), jnp.float32) for _ in range(L)],
    )(d3, f1s, f1bs, f2s, f2bs)


def _sc_body(x_hbm, wf_hbm, sd_hbm, zer_hbm, out_hbm,
             idx_v, wf_v, xg_v, agg_sh,
             sem_wf, sem_g, sem_sc):
    cid = lax.axis_index("c")
    sid = lax.axis_index("s")
    wid = cid * 16 + sid
    row0 = sid * RPS

    @pl.when(sid < 15)
    def _z0():
        pltpu.sync_copy(zer_hbm.at[pl.ds(row0, RPS)], agg_sh.at[pl.ds(row0, RPS)])

    @pl.when(sid == 15)
    def _z1():
        pltpu.sync_copy(zer_hbm.at[pl.ds(15 * RPS, RPS_LAST)],
                        agg_sh.at[pl.ds(15 * RPS, RPS_LAST)])

    plsc.subcore_barrier()

    def start_in(c, b):
        base = wid * EPW + c * CH
        pltpu.async_copy(wf_hbm.at[pl.ds(base, CH)], wf_v.at[b], sem_wf)
        pltpu.async_copy(sd_hbm.at[wid, c], idx_v.at[b], sem_wf)

    def wait_in(b):
        pltpu.make_async_copy(wf_hbm.at[pl.ds(0, CH)], wf_v.at[b], sem_wf).wait()
        pltpu.make_async_copy(sd_hbm.at[0, 0], idx_v.at[b], sem_wf).wait()

    def drain_sc(b):
        pltpu.make_async_copy(xg_v.at[b], agg_sh.at[idx_v.at[b, 1]], sem_sc).wait()

    def compute(b):
        def row(e, c2):
            for j in range(H // 16):
                sl = pl.ds(j * 16, 16)
                xg_v[b, e, sl] = xg_v[b, e, sl] * wf_v[b, e, sl]
            return c2

        lax.fori_loop(0, CH, row, 0)

    def chunk_step(c, b, first, last):
        wait_in(b)
        gather = pltpu.async_copy(x_hbm.at[idx_v.at[b, 0]], xg_v.at[b], sem_g)
        if not first:
            drain_sc(1 - b)

        if not last:
            @pl.when(c + 1 < NCH)
            def _pf():
                start_in(c + 1, 1 - b)

        gather.wait()
        compute(b)
        pltpu.async_copy(xg_v.at[b], agg_sh.at[idx_v.at[b, 1]], sem_sc, add=True)

    start_in(0, 0)

    def pair(g, carry):
        c0 = 2 * g

        @pl.when(g == 0)
        def _first():
            chunk_step(0, 0, True, False)

        @pl.when(g > 0)
        def _steady():
            chunk_step(c0, 0, False, False)

        chunk_step(c0 + 1, 1, False, False)
        return carry

    lax.fori_loop(0, NCH // 2, pair, 0)
    drain_sc(1)
    plsc.subcore_barrier()

    @pl.when(sid < 15)
    def _w0():
        pltpu.sync_copy(agg_sh.at[pl.ds(row0, RPS)],
                        out_hbm.at[pl.ds(cid * N + row0, RPS)])

    @pl.when(sid == 15)
    def _w1():
        pltpu.sync_copy(agg_sh.at[pl.ds(15 * RPS, RPS_LAST)],
                        out_hbm.at[pl.ds(cid * N + 15 * RPS, RPS_LAST)])


def _sc_layer(x, wf, sd, zeros_nh):
    mesh = plsc.VectorSubcoreMesh(core_axis_name="c", subcore_axis_name="s",
                                  num_cores=2, num_subcores=16)
    k = pl.kernel(
        _sc_body,
        out_type=jax.ShapeDtypeStruct((2 * N, H), jnp.float32),
        mesh=mesh,
        scratch_types=[
            pltpu.VMEM((2, 2, CH), jnp.int32),
            pltpu.VMEM((2, CH, H), jnp.float32),
            pltpu.VMEM((2, CH, H), jnp.float32),
            pltpu.VMEM_SHARED((N, H), jnp.float32),
            pltpu.SemaphoreType.DMA,
            pltpu.SemaphoreType.DMA,
            pltpu.SemaphoreType.DMA,
        ],
    )
    return k(x, wf, sd, zeros_nh)


def _update_body(x_ref, agg_ref, u1_ref, u1b_ref, u2_ref, u2b_ref, o_ref):
    agg = agg_ref[0] + agg_ref[1]
    t = _silu(jnp.dot(agg, u1_ref[...], preferred_element_type=jnp.float32)
              + u1b_ref[...])
    upd = jnp.dot(t, u2_ref[...], preferred_element_type=jnp.float32) + u2b_ref[...]
    o_ref[...] = x_ref[...] + upd


def _update(x, aggp, u1, u1b, u2, u2b):
    return pl.pallas_call(
        _update_body,
        grid=(N // BN,),
        in_specs=[
            pl.BlockSpec((BN, H), lambda i: (i, 0)),
            pl.BlockSpec((2, BN, H), lambda i: (0, i, 0)),
            pl.BlockSpec((H, H), lambda i: (0, 0)),
            pl.BlockSpec((1, H), lambda i: (0, 0)),
            pl.BlockSpec((H, H), lambda i: (0, 0)),
            pl.BlockSpec((1, H), lambda i: (0, 0)),
        ],
        out_specs=pl.BlockSpec((BN, H), lambda i: (i, 0)),
        out_shape=jax.ShapeDtypeStruct((N, H), jnp.float32),
    )(x, aggp, u1, u1b, u2, u2b)


def _readout_body(b_ref, x_ref, g1_ref, g1b_ref, g2_ref, g2b_ref,
                  h1_ref, h1b_ref, h2_ref, o_ref, ssum_ref, cnt_ref):
    i = pl.program_id(0)
    nblk = pl.num_programs(0)

    @pl.when(i == 0)
    def _init():
        ssum_ref[...] = jnp.zeros_like(ssum_ref)
        cnt_ref[...] = jnp.zeros_like(cnt_ref)

    bcol = b_ref[0]
    onehot = (bcol == lax.broadcasted_iota(jnp.int32, (BN, NG), 1))
    oh = onehot.astype(jnp.float32)
    ssum_ref[...] += lax.dot_general(oh, x_ref[...],
                                     (((0,), (0,)), ((), ())),
                                     preferred_element_type=jnp.float32,
                                     precision=lax.Precision.HIGHEST)
    cnt_ref[...] += lax.dot_general(oh, jnp.ones((BN, H), jnp.float32),
                                    (((0,), (0,)), ((), ())),
                                    preferred_element_type=jnp.float32,
                                    precision=lax.Precision.HIGHEST)

    @pl.when(i == nblk - 1)
    def _fin():
        g = ssum_ref[...] / jnp.maximum(cnt_ref[...], 1.0)
        t = _silu(jnp.dot(g, g1_ref[...], preferred_element_type=jnp.float32)
                  + g1b_ref[...])
        t = jnp.dot(t, g2_ref[...], preferred_element_type=jnp.float32) + g2b_ref[...]
        t = _silu(jnp.dot(t, h1_ref[...], preferred_element_type=jnp.float32)
                  + h1b_ref[...])
        o_ref[...] = jnp.dot(t, h2_ref[...], preferred_element_type=jnp.float32)


def _readout(b3, x, g1, g1b, g2, g2b, h1p, h1bp, h2p):
    return pl.pallas_call(
        _readout_body,
        grid=(N // BN,),
        in_specs=[
            pl.BlockSpec((1, BN, 1), lambda i: (i, 0, 0)),
            pl.BlockSpec((BN, H), lambda i: (i, 0)),
            pl.BlockSpec((H, H), lambda i: (0, 0)),
            pl.BlockSpec((1, H), lambda i: (0, 0)),
            pl.BlockSpec((H, H), lambda i: (0, 0)),
            pl.BlockSpec((1, H), lambda i: (0, 0)),
            pl.BlockSpec((H, H), lambda i: (0, 0)),
            pl.BlockSpec((1, H), lambda i: (0, 0)),
            pl.BlockSpec((H, H), lambda i: (0, 0)),
        ],
        out_specs=pl.BlockSpec((NG, H), lambda i: (0, 0)),
        out_shape=jax.ShapeDtypeStruct((NG, H), jnp.float32),
        scratch_shapes=[
            pltpu.VMEM((NG, H), jnp.float32),
            pltpu.VMEM((NG, H), jnp.float32),
        ],
    )(b3, x, g1, g1b, g2, g2b, h1p, h1bp, h2p)


def kernel(z, edge_index, edge_dist, batch, params):
    p = params
    src = edge_index[0].astype(jnp.int32)
    dst = edge_index[1].astype(jnp.int32)

    emb_pad = jnp.zeros((H, H), jnp.float32).at[:ZMAX].set(p['atom_emb'])
    z3 = z.astype(jnp.int32).reshape(N // BN, BN, 1)
    b3 = batch.astype(jnp.int32).reshape(N // BN, BN, 1)
    d3 = edge_dist.reshape(E // BE, BE, 1)

    sd = jnp.stack([src.reshape(NW, NCH, CH), dst.reshape(NW, NCH, CH)], axis=2)

    f1s = jnp.stack([lp['f1_w'] for lp in p['layers']])
    f1bs = jnp.stack([lp['f1_b'] for lp in p['layers']]).reshape(L, 1, H)
    f2s = jnp.stack([lp['f2_w'] for lp in p['layers']])
    f2bs = jnp.stack([lp['f2_b'] for lp in p['layers']]).reshape(L, 1, H)

    x = _embed(z3, emb_pad)
    wfs = _filters(d3, f1s, f1bs, f2s, f2bs)
    zeros_nh = jnp.zeros((N, H), jnp.float32)

    for l, lp in enumerate(p['layers']):
        aggp = _sc_layer(x, wfs[l], sd, zeros_nh)
        x = _update(x, aggp.reshape(2, N, H),
                    lp['u1_w'], lp['u1_b'].reshape(1, H),
                    lp['u2_w'], lp['u2_b'].reshape(1, H))

    h1p = jnp.zeros((H, H), jnp.float32).at[:, :H // 2].set(p['h1_w'])
    h1bp = jnp.zeros((1, H), jnp.float32).at[0, :H // 2].set(p['h1_b'])
    h2p = jnp.zeros((H, H), jnp.float32).at[:H // 2, 0].set(p['h2_w'][:, 0])
    out = _readout(b3, x,
                   p['g1_w'], p['g1_b'].reshape(1, H),
                   p['g2_w'], p['g2_b'].reshape(1, H),
                   h1p, h1bp, h2p)
    return out[:, 0] + p['h2_b'][0]
